# Initial kernel scaffold; baseline (speedup 1.0000x reference)
import jax
import jax.numpy as jnp
from jax.experimental import pallas as pl


def kernel(x, adj_t, W1, b1, g1, beta1, W2, b2, g2, beta2):
    raise NotImplementedError("write your pallas kernel here")


# speedup vs baseline: 9.8515x; 9.8515x over previous
import functools

import jax
import jax.numpy as jnp
from jax import lax
from jax.experimental import pallas as pl
from jax.experimental.pallas import tpu as pltpu
from jax.experimental.pallas import tpu_sc as plsc

N = 10000
NP = 10240
D = 128
GROUP = 128
NW = 32
ROWS_PER_SUB = NP // 16

@functools.cache
def _mesh():
    return plsc.VectorSubcoreMesh(core_axis_name="c", subcore_axis_name="s")


def _pad_edges(src, dst):
    e = src.shape[0]
    e_pad = ((e + NW * GROUP - 1) // (NW * GROUP)) * (NW * GROUP)
    pad = e_pad - e
    srcp = jnp.concatenate([src, jnp.zeros((pad,), jnp.int32)])
    dstp = jnp.concatenate([dst, jnp.full((pad,), N, jnp.int32)])
    return srcp, dstp, e_pad


def _make_deg_kernel(e_pad):
    per_w = e_pad // NW
    groups = per_w // GROUP

    @functools.partial(
        pl.kernel,
        out_type=jax.ShapeDtypeStruct((2 * NP, D), jnp.float32),
        mesh=_mesh(),
        scratch_types=[
            pltpu.VMEM((GROUP,), jnp.int32),
            pltpu.VMEM((GROUP, D), jnp.float32),
            pltpu.VMEM_SHARED((NP, D), jnp.float32),
        ],
    )
    def deg_kernel(dsts, zeros16, ones16, out, didx, obuf, acc):
        c = lax.axis_index("c")
        s = lax.axis_index("s")
        wid = s * 2 + c
        pltpu.sync_copy(ones16, obuf)
        pltpu.sync_copy(
            zeros16.at[pl.ds(s * ROWS_PER_SUB, ROWS_PER_SUB)],
            acc.at[pl.ds(s * ROWS_PER_SUB, ROWS_PER_SUB)],
        )
        plsc.subcore_barrier()
        base = wid * per_w

        def body(g, carry):
            off = base + g * GROUP
            pltpu.sync_copy(dsts.at[pl.ds(off, GROUP)], didx)
            pltpu.sync_copy(obuf, acc.at[didx], add=True)
            return carry

        lax.fori_loop(0, groups, body, 0)
        plsc.subcore_barrier()
        pltpu.sync_copy(
            acc.at[pl.ds(s * ROWS_PER_SUB, ROWS_PER_SUB)],
            out.at[pl.ds(c * NP + s * ROWS_PER_SUB, ROWS_PER_SUB)],
        )

    return deg_kernel


def _make_scatter_kernel(e_pad):
    per_w = e_pad // NW
    groups = per_w // GROUP

    @functools.partial(
        pl.kernel,
        out_type=jax.ShapeDtypeStruct((2 * NP, D), jnp.float32),
        mesh=_mesh(),
        scratch_types=[
            pltpu.VMEM((GROUP,), jnp.int32),
            pltpu.VMEM((GROUP,), jnp.int32),
            pltpu.VMEM((GROUP, D), jnp.float32),
            pltpu.VMEM_SHARED((NP, D), jnp.float32),
            pltpu.SemaphoreType.DMA,
        ],
    )
    def scat_kernel(table, srcs, dsts, zeros, out, sidx, didx, rows, acc, sem):
        c = lax.axis_index("c")
        s = lax.axis_index("s")
        wid = s * 2 + c
        pltpu.sync_copy(
            zeros.at[pl.ds(s * ROWS_PER_SUB, ROWS_PER_SUB)],
            acc.at[pl.ds(s * ROWS_PER_SUB, ROWS_PER_SUB)],
        )
        plsc.subcore_barrier()
        base = wid * per_w

        def body(g, carry):
            off = base + g * GROUP
            pltpu.sync_copy(srcs.at[pl.ds(off, GROUP)], sidx)
            pltpu.sync_copy(dsts.at[pl.ds(off, GROUP)], didx)
            pltpu.async_copy(table.at[sidx], rows, sem).wait()
            pltpu.sync_copy(rows, acc.at[didx], add=True)
            return carry

        lax.fori_loop(0, groups, body, 0)
        plsc.subcore_barrier()
        pltpu.sync_copy(
            acc.at[pl.ds(s * ROWS_PER_SUB, ROWS_PER_SUB)],
            out.at[pl.ds(c * NP + s * ROWS_PER_SUB, ROWS_PER_SUB)],
        )

    return scat_kernel


_BLK = 400
_GRID = N // _BLK


def _dinv(deg):
    return jnp.where(deg > 0, lax.rsqrt(deg), 0.0)


def _tc_pre_body(x_ref, w_ref, deg_ref, out_ref):
    h = jnp.dot(x_ref[...], w_ref[...], preferred_element_type=jnp.float32)
    out_ref[...] = h * _dinv(deg_ref[...])


def _tc_mid_body(p0_ref, p1_ref, hp_ref, deg_ref, b_ref, g_ref, bt_ref,
                 w_ref, out_ref):
    dinv = _dinv(deg_ref[...])
    u = dinv * (p0_ref[...] + p1_ref[...] + hp_ref[...]) + b_ref[...]
    mu = jnp.mean(u, axis=-1, keepdims=True)
    var = jnp.mean((u - mu) ** 2, axis=-1, keepdims=True)
    y = g_ref[...] * (u - mu) * lax.rsqrt(var + 1e-5) + bt_ref[...]
    f = jnp.maximum(y, 0.0)
    out_ref[...] = jnp.dot(f, w_ref[...],
                           preferred_element_type=jnp.float32) * dinv


def _tc_post_body(p0_ref, p1_ref, hp_ref, deg_ref, b_ref, g_ref, bt_ref,
                  out_ref):
    dinv = _dinv(deg_ref[...])
    u = dinv * (p0_ref[...] + p1_ref[...] + hp_ref[...]) + b_ref[...]
    mu = jnp.mean(u, axis=-1, keepdims=True)
    var = jnp.mean((u - mu) ** 2, axis=-1, keepdims=True)
    y = g_ref[...] * (u - mu) * lax.rsqrt(var + 1e-5) + bt_ref[...]
    out_ref[...] = jnp.maximum(y, 0.0)


_row_spec = pl.BlockSpec((_BLK, D), lambda i: (i, 0))
_w_spec = pl.BlockSpec((D, D), lambda i: (0, 0))
_deg_spec = pl.BlockSpec((_BLK, 1), lambda i: (i, 0))
_vec_spec = pl.BlockSpec((1, D), lambda i: (0, 0))


def _tc_pre(x, w, deg):
    return pl.pallas_call(
        _tc_pre_body,
        grid=(_GRID,),
        in_specs=[_row_spec, _w_spec, _deg_spec],
        out_specs=_row_spec,
        out_shape=jax.ShapeDtypeStruct((N, D), jnp.float32),
    )(x, w, deg)


def _tc_mid(p0, p1, hp, deg, b, g, bt, w):
    return pl.pallas_call(
        _tc_mid_body,
        grid=(_GRID,),
        in_specs=[_row_spec, _row_spec, _row_spec, _deg_spec,
                  _vec_spec, _vec_spec, _vec_spec, _w_spec],
        out_specs=_row_spec,
        out_shape=jax.ShapeDtypeStruct((N, D), jnp.float32),
    )(p0, p1, hp, deg, b, g, bt, w)


def _tc_post(p0, p1, hp, deg, b, g, bt):
    return pl.pallas_call(
        _tc_post_body,
        grid=(_GRID,),
        in_specs=[_row_spec, _row_spec, _row_spec, _deg_spec,
                  _vec_spec, _vec_spec, _vec_spec],
        out_specs=_row_spec,
        out_shape=jax.ShapeDtypeStruct((N, D), jnp.float32),
    )(p0, p1, hp, deg, b, g, bt)


def kernel(x, adj_t, W1, b1, g1, beta1, W2, b2, g2, beta2):
    src = adj_t[0].astype(jnp.int32)
    dst = adj_t[1].astype(jnp.int32)
    srcp, dstp, e_pad = _pad_edges(src, dst)

    ones16 = jnp.ones((GROUP, D), jnp.float32)
    zeros = jnp.zeros((NP, D), jnp.float32)

    deg16 = _make_deg_kernel(e_pad)(dstp, zeros, ones16)
    deg = (deg16[:N, 0] + deg16[NP:NP + N, 0] + 1.0).reshape(N, 1)

    b1r, g1r, bt1 = b1.reshape(1, D), g1.reshape(1, D), beta1.reshape(1, D)
    b2r, g2r, bt2 = b2.reshape(1, D), g2.reshape(1, D), beta2.reshape(1, D)

    scat = _make_scatter_kernel(e_pad)
    hp1 = _tc_pre(x, W1, deg)
    p1 = scat(hp1, srcp, dstp, zeros)
    hp2 = _tc_mid(p1[:N], p1[NP:NP + N], hp1, deg, b1r, g1r, bt1, W2)
    p2 = scat(hp2, srcp, dstp, zeros)
    return _tc_post(p2[:N], p2[NP:NP + N], hp2, deg, b2r, g2r, bt2)
